# baseline (speedup 1.0000x reference)
import functools

import jax
import jax.numpy as jnp
from jax import lax
from jax.experimental import pallas as pl
from jax.experimental.pallas import tpu as pltpu
from jax.experimental.pallas import tpu_sc as plsc

_INFO = plsc.get_sparse_core_info()
_NC = _INFO.num_cores
_NS = _INFO.num_subcores
_L = _INFO.num_lanes
_NW = _NC * _NS

_W = 128


def _make_kernel(n_e, s, d):
    assert n_e % (_W * _NW) == 0 or n_e == _W * _NW
    assert s % 2 == 0 and d % _L == 0

    mesh = plsc.VectorSubcoreMesh(core_axis_name="c", subcore_axis_name="s")

    @functools.partial(
        pl.kernel,
        mesh=mesh,
        compiler_params=pltpu.CompilerParams(
            use_tc_tiling_on_sc=False, needs_layout_passes=False
        ),
        out_type=jax.ShapeDtypeStruct((s * d, n_e), jnp.float32),
        scratch_types=[
            pltpu.VMEM((s, _W), jnp.int32),
            pltpu.VMEM((2, _W, d), jnp.float32),
            pltpu.VMEM((2, d, _W + 5), jnp.float32),
            pltpu.SemaphoreType.DMA,
            pltpu.SemaphoreType.DMA,
            pltpu.SemaphoreType.DMA,
            pltpu.SemaphoreType.DMA,
        ],
    )
    def emb(idx_hbm, table_hbm, out_hbm, idx_wv, rows_v, slab_v,
            g0, g1, o0, o1):
        wid = lax.axis_index("s") * _NC + lax.axis_index("c")
        i0 = wid * _W
        gsem = (g0, g1)
        osem = (o0, o1)
        iota16 = lax.iota(jnp.int32, _L)

        pltpu.sync_copy(idx_hbm.at[:, pl.ds(i0, _W)], idx_wv)

        def fire_gather(j, b):
            pltpu.async_copy(
                table_hbm.at[idx_wv.at[j]], rows_v.at[b], gsem[b]
            )

        def drain_gather(j, b):
            pltpu.make_async_copy(
                table_hbm.at[idx_wv.at[j]], rows_v.at[b], gsem[b]
            ).wait()

        def fire_wb(j, b):
            pltpu.async_copy(
                slab_v.at[b, :, pl.ds(0, _W)],
                out_hbm.at[pl.ds(j * d, d), pl.ds(i0, _W)],
                osem[b],
            )

        def drain_wb(j, b):
            pltpu.make_async_copy(
                slab_v.at[b, :, pl.ds(0, _W)],
                out_hbm.at[pl.ds(j * d, d), pl.ds(i0, _W)],
                osem[b],
            ).wait()

        dd_rows = [dd16 * _L + iota16 for dd16 in range(d // _L)]

        def transpose_mask(j, b):
            @plsc.parallel_loop(0, _W // _L, 1, unroll=2)
            def _(i16):
                mvec = idx_wv[j, pl.ds(i16 * _L, _L)]
                fvec = jnp.minimum(mvec, 1).astype(jnp.float32)
                for r in range(_L):
                    fac = jnp.full((_L,), fvec[r], jnp.float32)
                    i = i16 * _L + r
                    col = jnp.full((_L,), i, jnp.int32)
                    for dd16 in range(d // _L):
                        v = rows_v[b, i, pl.ds(dd16 * _L, _L)]
                        plsc.store_scatter(
                            slab_v.at[b], [dd_rows[dd16], col], v * fac
                        )

        fire_gather(0, 0)

        def body(jj, carry):
            for b in (0, 1):
                j = jj * 2 + b

                @pl.when(j + 1 < s)
                def _():
                    fire_gather(j + 1, 1 - b)

                drain_gather(j, b)

                @pl.when(j >= 2)
                def _():
                    drain_wb(j - 2, b)

                transpose_mask(j, b)
                fire_wb(j, b)
            return carry

        lax.fori_loop(0, s // 2, body, 0)
        drain_wb(s - 2, 0)
        drain_wb(s - 1, 1)

    return emb


def kernel(embedding_property, table):
    n_e, s = embedding_property.shape
    n_emb, d = table.shape
    idx_t = jnp.transpose(embedding_property).astype(jnp.int32)
    out2d = _make_kernel(n_e, s, d)(idx_t, table)
    return jnp.transpose(out2d.reshape(s, d, n_e), (2, 0, 1))


# speedup vs baseline: 1.3928x; 1.3928x over previous
import functools

import jax
import jax.numpy as jnp
from jax import lax
from jax.experimental import pallas as pl
from jax.experimental.pallas import tpu as pltpu
from jax.experimental.pallas import tpu_sc as plsc

_INFO = plsc.get_sparse_core_info()
_NC = _INFO.num_cores
_NS = _INFO.num_subcores
_L = _INFO.num_lanes
_NW = _NC * _NS

_IDX_PER_STREAM = 128
_K_PER_CHUNK = 5
_CHUNK = _K_PER_CHUNK * _IDX_PER_STREAM
_NBUF = 2


def _make_kernel(n_idx, d):
    per_w = n_idx // _NW
    n_chunks = per_w // _CHUNK
    assert per_w % _CHUNK == 0 and n_idx % _NW == 0

    mesh = plsc.VectorSubcoreMesh(core_axis_name="c", subcore_axis_name="s")

    @functools.partial(
        pl.kernel,
        mesh=mesh,
        compiler_params=pltpu.CompilerParams(use_tc_tiling_on_sc=False),
        out_type=jax.ShapeDtypeStruct((n_idx, d), jnp.float32),
        scratch_types=[
            pltpu.VMEM((per_w,), jnp.int32),
            pltpu.VMEM((_NBUF, _CHUNK, d), jnp.float32),
            pltpu.SemaphoreType.DMA,
            pltpu.SemaphoreType.DMA,
            pltpu.SemaphoreType.DMA,
            pltpu.SemaphoreType.DMA,
        ],
    )
    def emb(idx_hbm, table_hbm, out_hbm, idx_v, rows_v, g0, g1, o0, o1):
        wid = lax.axis_index("s") * _NC + lax.axis_index("c")
        out_base = wid * per_w
        gsem = (g0, g1)
        osem = (o0, o1)

        pltpu.sync_copy(idx_hbm.at[pl.ds(out_base, per_w)], idx_v)

        def fire_gathers(c, b):
            for k in range(_K_PER_CHUNK):
                pltpu.async_copy(
                    table_hbm.at[
                        idx_v.at[
                            pl.ds(c * _CHUNK + k * _IDX_PER_STREAM,
                                  _IDX_PER_STREAM)
                        ]
                    ],
                    rows_v.at[
                        b, pl.ds(k * _IDX_PER_STREAM, _IDX_PER_STREAM)
                    ],
                    gsem[b],
                )

        def drain_gathers(c, b):
            for k in range(_K_PER_CHUNK):
                pltpu.make_async_copy(
                    table_hbm.at[
                        idx_v.at[
                            pl.ds(c * _CHUNK + k * _IDX_PER_STREAM,
                                  _IDX_PER_STREAM)
                        ]
                    ],
                    rows_v.at[
                        b, pl.ds(k * _IDX_PER_STREAM, _IDX_PER_STREAM)
                    ],
                    gsem[b],
                ).wait()

        def fix_chunk(c, b):
            def vmin_step(j, m):
                return jnp.minimum(
                    m, idx_v[pl.ds(c * _CHUNK + j * _L, _L)]
                )

            m0 = idx_v[pl.ds(c * _CHUNK, _L)]
            mv = lax.fori_loop(1, _CHUNK // _L, vmin_step, m0)
            smin = mv[0]
            for r in range(1, _L):
                smin = jnp.minimum(smin, mv[r])

            @pl.when(smin == 0)
            def _():
                def fix_group(g, carry):
                    m16 = idx_v[pl.ds(c * _CHUNK + g * _L, _L)]
                    gmin = m16[0]
                    for r in range(1, _L):
                        gmin = jnp.minimum(gmin, m16[r])

                    @pl.when(gmin == 0)
                    def _():
                        for r in range(_L):
                            sf = jnp.minimum(m16[r], 1).astype(jnp.float32)
                            fac = jnp.full((_L,), sf, jnp.float32)
                            row = g * _L + r
                            for cc in range(d // _L):
                                v = rows_v[b, row, pl.ds(cc * _L, _L)]
                                rows_v[b, row, pl.ds(cc * _L, _L)] = v * fac

                    return carry

                lax.fori_loop(0, _CHUNK // _L, fix_group, 0)

        def fire_writeback(c, b):
            pltpu.async_copy(
                rows_v.at[b],
                out_hbm.at[pl.ds(out_base + c * _CHUNK, _CHUNK)],
                osem[b],
            )

        def drain_writeback(c, b):
            pltpu.make_async_copy(
                rows_v.at[b],
                out_hbm.at[pl.ds(out_base + c * _CHUNK, _CHUNK)],
                osem[b],
            ).wait()

        for c in range(n_chunks):
            b = c % _NBUF
            if c >= _NBUF:
                drain_writeback(c - _NBUF, b)
            fire_gathers(c, b)
            if c >= 1:
                pb = (c - 1) % _NBUF
                drain_gathers(c - 1, pb)
                fix_chunk(c - 1, pb)
                fire_writeback(c - 1, pb)
        last = n_chunks - 1
        lb = last % _NBUF
        drain_gathers(last, lb)
        fix_chunk(last, lb)
        fire_writeback(last, lb)
        drain_writeback(last - 1, (last - 1) % _NBUF)
        drain_writeback(last, lb)

    return emb


def kernel(embedding_property, table):
    b, s = embedding_property.shape
    n_emb, d = table.shape
    n_idx = b * s
    idx_flat = embedding_property.reshape(n_idx).astype(jnp.int32)
    out = _make_kernel(n_idx, d)(idx_flat, table)
    return (
        out.reshape(b, s * d)
        .transpose()
        .reshape(s, d, b)
        .transpose(2, 0, 1)
    )
